# baseline (device time: 241637 ns/iter reference)
import jax
import jax.numpy as jnp
from jax import lax
from jax.experimental import pallas as pl
from jax.experimental.pallas import tpu as pltpu

N_DEV = 32
SQ = 1024
SKV = 1024
H_LOC = 8
DH = 128
D_MODEL = 1024
SCALE = 0.08838834764831843
CHUNK = SQ // N_DEV
N_HOPS = 2 * (N_DEV - 1)


def _compute_body(x_ref, wq_ref, k_ref, v_ref, wo_ref, out_ref):
    q = jnp.dot(x_ref[...], wq_ref[...], preferred_element_type=jnp.float32)

    ri = lax.broadcasted_iota(jnp.int32, (SQ, SKV), 0)
    ci = lax.broadcasted_iota(jnp.int32, (SQ, SKV), 1)
    mask = ((ri // 64) % 4) == ((ci // 64) % 4)

    parts = []
    for h in range(H_LOC):
        qh = q[:, h * DH:(h + 1) * DH]
        kh = k_ref[h]
        vh = v_ref[h]
        s = lax.dot_general(
            qh, kh, (((1,), (1,)), ((), ())),
            preferred_element_type=jnp.float32,
        ) * SCALE
        s = jnp.where(mask, s, -1e9)
        m = jnp.max(s, axis=1, keepdims=True)
        w = jnp.exp(s - m)
        w = w / jnp.sum(w, axis=1, keepdims=True)
        parts.append(jnp.dot(w, vh, preferred_element_type=jnp.float32))
    ctx = jnp.concatenate(parts, axis=1)

    partial = jnp.dot(ctx, wo_ref[...], preferred_element_type=jnp.float32)
    out_ref[...] = partial.reshape(N_DEV, CHUNK, D_MODEL)


def _allreduce_body(part_ref, out_ref, comm_ref, send_sems, recv_sems):
    my = lax.axis_index("i")
    left = (my - 1) % N_DEV
    right = (my + 1) % N_DEV

    out_ref[...] = part_ref[...]

    barrier = pltpu.get_barrier_semaphore()
    for nbr in (left, right):
        pl.semaphore_signal(
            barrier, inc=1, device_id=(nbr,),
            device_id_type=pl.DeviceIdType.MESH,
        )
    pl.semaphore_wait(barrier, 2)

    def rs_step(s, carry):
        send_idx = (my - s) % N_DEV
        recv_idx = (my - s - 1) % N_DEV
        rdma = pltpu.make_async_remote_copy(
            src_ref=out_ref.at[send_idx],
            dst_ref=comm_ref.at[s],
            send_sem=send_sems.at[s],
            recv_sem=recv_sems.at[s],
            device_id=(right,),
            device_id_type=pl.DeviceIdType.MESH,
        )
        rdma.start()
        rdma.wait()
        out_ref[recv_idx] = out_ref[recv_idx] + comm_ref[s]
        return carry

    lax.fori_loop(0, N_DEV - 1, rs_step, 0)

    def ag_step(t, carry):
        slot = N_DEV - 1 + t
        send_idx = (my + 1 - t) % N_DEV
        recv_idx = (my - t) % N_DEV
        rdma = pltpu.make_async_remote_copy(
            src_ref=out_ref.at[send_idx],
            dst_ref=comm_ref.at[slot],
            send_sem=send_sems.at[slot],
            recv_sem=recv_sems.at[slot],
            device_id=(right,),
            device_id_type=pl.DeviceIdType.MESH,
        )
        rdma.start()
        rdma.wait()
        out_ref[recv_idx] = comm_ref[slot]
        return carry

    lax.fori_loop(0, N_DEV - 1, ag_step, 0)


def kernel(x, Wq, K_ext, V_ext, Wo):
    my = lax.axis_index("i")

    k_loc = lax.dynamic_slice_in_dim(K_ext[0], my * H_LOC, H_LOC, axis=1)
    v_loc = lax.dynamic_slice_in_dim(V_ext[0], my * H_LOC, H_LOC, axis=1)
    k_loc = jnp.transpose(k_loc, (1, 0, 2))
    v_loc = jnp.transpose(v_loc, (1, 0, 2))

    partial = pl.pallas_call(
        _compute_body,
        out_shape=jax.ShapeDtypeStruct((N_DEV, CHUNK, D_MODEL), jnp.float32),
        in_specs=[pl.BlockSpec(memory_space=pltpu.VMEM)] * 5,
        out_specs=pl.BlockSpec(memory_space=pltpu.VMEM),
    )(x[0], Wq, k_loc, v_loc, Wo)

    out = pl.pallas_call(
        _allreduce_body,
        out_shape=jax.ShapeDtypeStruct((N_DEV, CHUNK, D_MODEL), jnp.float32),
        in_specs=[pl.BlockSpec(memory_space=pltpu.VMEM)],
        out_specs=pl.BlockSpec(memory_space=pltpu.VMEM),
        scratch_shapes=[
            pltpu.VMEM((N_HOPS, CHUNK, D_MODEL), jnp.float32),
            pltpu.SemaphoreType.DMA((N_HOPS,)),
            pltpu.SemaphoreType.DMA((N_HOPS,)),
        ],
        compiler_params=pltpu.CompilerParams(collective_id=0),
    )(partial)

    return out.reshape(1, SQ, D_MODEL)


# device time: 164132 ns/iter; 1.4722x vs baseline; 1.4722x over previous
import jax
import jax.numpy as jnp
from jax import lax
from jax.experimental import pallas as pl
from jax.experimental.pallas import tpu as pltpu

N_DEV = 32
SQ = 1024
SKV = 1024
H_LOC = 8
DH = 128
D_MODEL = 1024
SCALE = 0.08838834764831843
CHUNK = SQ // N_DEV
N_HOPS = 2 * (N_DEV - 1)


def _compute_body(x_ref, wq_ref, k_ref, v_ref, wo_ref, out_ref):
    q = jnp.dot(x_ref[...], wq_ref[...], preferred_element_type=jnp.float32)

    ri = lax.broadcasted_iota(jnp.int32, (SQ, SKV), 0)
    ci = lax.broadcasted_iota(jnp.int32, (SQ, SKV), 1)
    mask = ((ri // 64) % 4) == ((ci // 64) % 4)

    parts = []
    for h in range(H_LOC):
        qh = q[:, h * DH:(h + 1) * DH]
        kh = k_ref[h]
        vh = v_ref[h]
        s = lax.dot_general(
            qh, kh, (((1,), (1,)), ((), ())),
            preferred_element_type=jnp.float32,
        ) * SCALE
        s = jnp.where(mask, s, -1e9)
        m = jnp.max(s, axis=1, keepdims=True)
        w = jnp.exp(s - m)
        w = w / jnp.sum(w, axis=1, keepdims=True)
        parts.append(jnp.dot(w, vh, preferred_element_type=jnp.float32))
    ctx = jnp.concatenate(parts, axis=1)

    partial = jnp.dot(ctx, wo_ref[...], preferred_element_type=jnp.float32)
    out_ref[...] = partial.reshape(N_DEV, CHUNK, D_MODEL)


GP = 8
NG = 4
SUB = N_DEV // GP


def _allreduce_body(part_ref, out_ref, comm_a, comm_b, comm_c,
                    sa_s, sa_r, sb_s, sb_r, sc_s, sc_r):
    my = lax.axis_index("i")
    g = my // GP
    q = my % GP
    intra_right = g * GP + (q + 1) % GP
    intra_left = g * GP + (q - 1) % GP
    inter_right = ((g + 1) % NG) * GP + q
    inter_left = ((g - 1) % NG) * GP + q

    out_ref[...] = part_ref[...]

    barrier = pltpu.get_barrier_semaphore()
    for nbr in (intra_left, intra_right, inter_left, inter_right):
        pl.semaphore_signal(
            barrier, inc=1, device_id=(nbr,),
            device_id_type=pl.DeviceIdType.MESH,
        )
    pl.semaphore_wait(barrier, 4)

    def a_step(a, carry):
        send_c = (q - a) % GP
        recv_c = (q - a - 1) % GP
        rdma = pltpu.make_async_remote_copy(
            src_ref=out_ref.at[pl.ds(4 * send_c, 4)],
            dst_ref=comm_a.at[a],
            send_sem=sa_s.at[a],
            recv_sem=sa_r.at[a],
            device_id=(intra_right,),
            device_id_type=pl.DeviceIdType.MESH,
        )
        rdma.start()
        rdma.wait()
        out_ref[pl.ds(4 * recv_c, 4)] = out_ref[pl.ds(4 * recv_c, 4)] + comm_a[a]
        return carry

    lax.fori_loop(0, GP - 1, a_step, 0)

    base = 4 * ((q + 1) % GP)

    def b_rs_step(b, carry):
        send_s = base + (g - b) % NG
        recv_s = base + (g - b - 1) % NG
        rdma = pltpu.make_async_remote_copy(
            src_ref=out_ref.at[send_s],
            dst_ref=comm_b.at[b],
            send_sem=sb_s.at[b],
            recv_sem=sb_r.at[b],
            device_id=(inter_right,),
            device_id_type=pl.DeviceIdType.MESH,
        )
        rdma.start()
        rdma.wait()
        out_ref[recv_s] = out_ref[recv_s] + comm_b[b]
        return carry

    lax.fori_loop(0, NG - 1, b_rs_step, 0)

    def b_ag_step(b, carry):
        slot = NG - 1 + b
        send_s = base + (g + 1 - b) % NG
        recv_s = base + (g - b) % NG
        rdma = pltpu.make_async_remote_copy(
            src_ref=out_ref.at[send_s],
            dst_ref=comm_b.at[slot],
            send_sem=sb_s.at[slot],
            recv_sem=sb_r.at[slot],
            device_id=(inter_right,),
            device_id_type=pl.DeviceIdType.MESH,
        )
        rdma.start()
        rdma.wait()
        out_ref[recv_s] = comm_b[slot]
        return carry

    lax.fori_loop(0, NG - 1, b_ag_step, 0)

    def c_step(t, carry):
        send_c = (q + 1 - t) % GP
        recv_c = (q - t) % GP
        rdma = pltpu.make_async_remote_copy(
            src_ref=out_ref.at[pl.ds(4 * send_c, 4)],
            dst_ref=comm_c.at[t],
            send_sem=sc_s.at[t],
            recv_sem=sc_r.at[t],
            device_id=(intra_right,),
            device_id_type=pl.DeviceIdType.MESH,
        )
        rdma.start()
        rdma.wait()
        out_ref[pl.ds(4 * recv_c, 4)] = comm_c[t]
        return carry

    lax.fori_loop(0, GP - 1, c_step, 0)


def kernel(x, Wq, K_ext, V_ext, Wo):
    my = lax.axis_index("i")

    k_loc = lax.dynamic_slice_in_dim(K_ext[0], my * H_LOC, H_LOC, axis=1)
    v_loc = lax.dynamic_slice_in_dim(V_ext[0], my * H_LOC, H_LOC, axis=1)
    k_loc = jnp.transpose(k_loc, (1, 0, 2))
    v_loc = jnp.transpose(v_loc, (1, 0, 2))

    partial = pl.pallas_call(
        _compute_body,
        out_shape=jax.ShapeDtypeStruct((N_DEV, CHUNK, D_MODEL), jnp.float32),
        in_specs=[pl.BlockSpec(memory_space=pltpu.VMEM)] * 5,
        out_specs=pl.BlockSpec(memory_space=pltpu.VMEM),
    )(x[0], Wq, k_loc, v_loc, Wo)

    out = pl.pallas_call(
        _allreduce_body,
        out_shape=jax.ShapeDtypeStruct((N_DEV, CHUNK, D_MODEL), jnp.float32),
        in_specs=[pl.BlockSpec(memory_space=pltpu.VMEM)],
        out_specs=pl.BlockSpec(memory_space=pltpu.VMEM),
        scratch_shapes=[
            pltpu.VMEM((GP - 1, 4, CHUNK, D_MODEL), jnp.float32),
            pltpu.VMEM((2 * (NG - 1), CHUNK, D_MODEL), jnp.float32),
            pltpu.VMEM((GP - 1, 4, CHUNK, D_MODEL), jnp.float32),
            pltpu.SemaphoreType.DMA((GP - 1,)),
            pltpu.SemaphoreType.DMA((GP - 1,)),
            pltpu.SemaphoreType.DMA((2 * (NG - 1),)),
            pltpu.SemaphoreType.DMA((2 * (NG - 1),)),
            pltpu.SemaphoreType.DMA((GP - 1,)),
            pltpu.SemaphoreType.DMA((GP - 1,)),
        ],
        compiler_params=pltpu.CompilerParams(collective_id=0),
    )(partial)

    return out.reshape(1, SQ, D_MODEL)
